# baseline (speedup 1.0000x reference)
import functools

import jax
import jax.numpy as jnp
from jax.experimental import pallas as pl
from jax.experimental.pallas import tpu as pltpu
from jax.experimental.pallas import tpu_sc as plsc

TOP_K = 2
BM = 128
SC_W = 128


def _sc_gather(src, idx):
    m = idx.shape[0]
    width = src.shape[1]
    idx2 = idx.reshape(1, m)
    mesh = plsc.VectorSubcoreMesh(core_axis_name="core",
                                  subcore_axis_name="subcore")

    @functools.partial(
        pl.kernel,
        out_type=jax.ShapeDtypeStruct((m, width), src.dtype),
        mesh=mesh)
    def kern(x_hbm, i_hbm, o_hbm):
        def body(i_vmem, o_vmem):
            pltpu.sync_copy(x_hbm.at[i_vmem.at[0]], o_vmem)

        pltpu.emit_pipeline(
            body,
            grid=(m // SC_W,),
            in_specs=[pl.BlockSpec((1, SC_W), index_map=lambda i: (0, i))],
            out_specs=[pl.BlockSpec((SC_W, width),
                                    index_map=lambda i: (i, 0))],
            core_axis_name=("core", "subcore"),
            dimension_semantics=(pltpu.PARALLEL,),
        )(i_hbm, o_hbm)

    return kern(src, idx2)


def _router_kernel(x_ref, gw_ref, i0_ref, i1_ref, w0_ref, w1_ref, psum_ref,
                   acc_ref):
    g = pl.program_id(0)
    logits = jax.lax.dot_general(
        x_ref[...].astype(jnp.bfloat16), gw_ref[...].astype(jnp.bfloat16),
        (((1,), (0,)), ((), ())),
        preferred_element_type=jnp.float32)
    m1 = jnp.max(logits, axis=-1, keepdims=True)
    i1 = jnp.argmax(logits, axis=-1).astype(jnp.int32)[:, None]
    lane = jax.lax.broadcasted_iota(jnp.int32, logits.shape, 1)
    masked = jnp.where(lane == i1, -jnp.inf, logits)
    m2 = jnp.max(masked, axis=-1, keepdims=True)
    i2 = jnp.argmax(masked, axis=-1).astype(jnp.int32)[:, None]
    e2 = jnp.exp(m2 - m1)
    i0_ref[...] = i1
    i1_ref[...] = i2
    w0_ref[...] = 1.0 / (1.0 + e2)
    w1_ref[...] = e2 / (1.0 + e2)
    p = jnp.exp(logits - m1)
    p = p / jnp.sum(p, axis=-1, keepdims=True)

    @pl.when(g == 0)
    def _():
        acc_ref[...] = jnp.zeros_like(acc_ref)

    acc_ref[...] += jnp.sum(p, axis=0, keepdims=True)

    @pl.when(g == pl.num_programs(0) - 1)
    def _():
        psum_ref[...] = acc_ref[...]


def _ffn_kernel(sp_ref, xs_ref, wcol_ref, wg_hbm, wu_hbm, wd_hbm, y_ref,
                wg_v, wu_v, wd_v, sems, *, nb):
    g = pl.program_id(0)
    e = sp_ref[g]
    e_prev = sp_ref[jnp.maximum(g - 1, 0)]
    nvalid = sp_ref[nb]

    @pl.when((g == 0) | (e != e_prev))
    def _():
        c1 = pltpu.make_async_copy(wg_hbm.at[e], wg_v, sems.at[0])
        c2 = pltpu.make_async_copy(wu_hbm.at[e], wu_v, sems.at[1])
        c3 = pltpu.make_async_copy(wd_hbm.at[e], wd_v, sems.at[2])
        c1.start()
        c2.start()
        c3.start()
        c1.wait()
        c2.wait()
        c3.wait()

    @pl.when(g < nvalid)
    def _():
        xs = xs_ref[...].astype(jnp.bfloat16)
        g1 = jax.lax.dot_general(xs, wg_v[...], (((1,), (0,)), ((), ())),
                                 preferred_element_type=jnp.float32)
        g2 = jax.lax.dot_general(xs, wu_v[...], (((1,), (0,)), ((), ())),
                                 preferred_element_type=jnp.float32)
        h = (g1 * jax.nn.sigmoid(g1) * g2).astype(jnp.bfloat16)
        y = jax.lax.dot_general(h, wd_v[...], (((1,), (0,)), ((), ())),
                                preferred_element_type=jnp.float32)
        y_ref[...] = (y * wcol_ref[...]).astype(jnp.bfloat16)


def _sum2_kernel(y3_ref, o_ref):
    o_ref[...] = jnp.sum(y3_ref[...].astype(jnp.float32), axis=1)


def kernel(x, gate_w, w_gate, w_up, w_down):
    b, s, d = x.shape
    e_num = gate_w.shape[1]
    f = w_gate.shape[2]
    t = b * s
    a = t * TOP_K
    a_pad = a + e_num * BM
    nb = a_pad // BM

    xf = x.reshape(t, d)
    wg = w_gate.astype(jnp.bfloat16)
    wu = w_up.astype(jnp.bfloat16)
    wd = w_down.astype(jnp.bfloat16)

    bmr = min(512, t)
    i0, i1, w0, w1, psum = pl.pallas_call(
        _router_kernel,
        grid=(t // bmr,),
        in_specs=[
            pl.BlockSpec((bmr, d), lambda g: (g, 0)),
            pl.BlockSpec((d, e_num), lambda g: (0, 0)),
        ],
        out_specs=[
            pl.BlockSpec((bmr, 1), lambda g: (g, 0)),
            pl.BlockSpec((bmr, 1), lambda g: (g, 0)),
            pl.BlockSpec((bmr, 1), lambda g: (g, 0)),
            pl.BlockSpec((bmr, 1), lambda g: (g, 0)),
            pl.BlockSpec((1, e_num), lambda g: (0, 0)),
        ],
        out_shape=[
            jax.ShapeDtypeStruct((t, 1), jnp.int32),
            jax.ShapeDtypeStruct((t, 1), jnp.int32),
            jax.ShapeDtypeStruct((t, 1), jnp.float32),
            jax.ShapeDtypeStruct((t, 1), jnp.float32),
            jax.ShapeDtypeStruct((1, e_num), jnp.float32),
        ],
        scratch_shapes=[pltpu.VMEM((1, e_num), jnp.float32)],
        compiler_params=pltpu.CompilerParams(
            dimension_semantics=("arbitrary",)),
    )(xf, gate_w)

    idx = jnp.concatenate([i0, i1], axis=1)
    rw = jnp.concatenate([w0, w1], axis=1)
    e_a = idx.reshape(-1)
    onehot = (e_a[:, None] == jnp.arange(e_num, dtype=jnp.int32)[None, :])
    cum = jnp.cumsum(onehot.astype(jnp.int32), axis=0)
    rank = jnp.take_along_axis(cum, e_a[:, None], axis=1)[:, 0] - 1
    counts = cum[-1]
    padded = ((counts + BM - 1) // BM) * BM
    ends = jnp.cumsum(padded)
    offsets = ends - padded
    pos = (offsets[e_a] + rank).astype(jnp.int32)
    tok = (jnp.arange(a, dtype=jnp.int32) // TOP_K)
    tok_of_pos = jnp.zeros((a_pad,), jnp.int32).at[pos].set(tok)
    w_of_pos = jnp.zeros((a_pad,), jnp.float32).at[pos].set(rw.reshape(-1))
    nvalid = (ends[-1] // BM).astype(jnp.int32)
    blk_start = jnp.arange(nb, dtype=jnp.int32) * BM
    block_expert = jnp.searchsorted(ends, blk_start, side='right')
    block_expert = jnp.minimum(block_expert, e_num - 1).astype(jnp.int32)
    sp = jnp.concatenate([block_expert, nvalid[None]])

    lane_d = min(256, d)
    sub = d // lane_d
    sub_i = jnp.arange(sub, dtype=jnp.int32)
    idx_disp = (tok_of_pos[:, None] * sub + sub_i).reshape(-1)
    xs = _sc_gather(xf.reshape(t * sub, lane_d), idx_disp).reshape(a_pad, d)

    grid_spec = pltpu.PrefetchScalarGridSpec(
        num_scalar_prefetch=1,
        grid=(nb,),
        in_specs=[
            pl.BlockSpec((BM, d), lambda g, sp_: (g, 0)),
            pl.BlockSpec((BM, 1), lambda g, sp_: (g, 0)),
            pl.BlockSpec(memory_space=pl.ANY),
            pl.BlockSpec(memory_space=pl.ANY),
            pl.BlockSpec(memory_space=pl.ANY),
        ],
        out_specs=pl.BlockSpec((BM, d), lambda g, sp_: (g, 0)),
        scratch_shapes=[
            pltpu.VMEM((d, f), jnp.bfloat16),
            pltpu.VMEM((d, f), jnp.bfloat16),
            pltpu.VMEM((f, d), jnp.bfloat16),
            pltpu.SemaphoreType.DMA((3,)),
        ],
    )
    y = pl.pallas_call(
        functools.partial(_ffn_kernel, nb=nb),
        grid_spec=grid_spec,
        out_shape=jax.ShapeDtypeStruct((a_pad, d), jnp.bfloat16),
        compiler_params=pltpu.CompilerParams(
            dimension_semantics=("arbitrary",)),
    )(sp, xs, w_of_pos[:, None], wg, wu, wd)

    y32 = jax.lax.bitcast_convert_type(
        y.reshape(a_pad, d // 2, 2), jnp.int32)
    lane_c = min(256, d // 2)
    sub_c = d // 2 // lane_c
    sub_ci = jnp.arange(sub_c, dtype=jnp.int32)
    idx_comb = (pos[:, None] * sub_c + sub_ci).reshape(-1)
    y01_32 = _sc_gather(y32.reshape(a_pad * sub_c, lane_c), idx_comb)
    y3 = jax.lax.bitcast_convert_type(
        y01_32.reshape(t, TOP_K, d // 2), jnp.bfloat16).reshape(t, TOP_K, d)
    bmc = min(512, t)
    out = pl.pallas_call(
        _sum2_kernel,
        grid=(t // bmc,),
        in_specs=[pl.BlockSpec((bmc, TOP_K, d), lambda g: (g, 0, 0))],
        out_specs=pl.BlockSpec((bmc, d), lambda g: (g, 0)),
        out_shape=jax.ShapeDtypeStruct((t, d), jnp.float32),
    )(y3)

    output = out.reshape(b, s, d)
    probs_mean = psum[0] / t
    frac = counts.astype(jnp.float32) / a
    aux = jnp.sum(probs_mean * frac) * e_num
    return output, aux


# speedup vs baseline: 1.0027x; 1.0027x over previous
import functools

import jax
import jax.numpy as jnp
from jax.experimental import pallas as pl
from jax.experimental.pallas import tpu as pltpu
from jax.experimental.pallas import tpu_sc as plsc

TOP_K = 2
BM = 128
SC_W = 128


def _sc_gather(src, idx):
    m = idx.shape[0]
    width = src.shape[1]
    idx2 = idx.reshape(1, m)
    mesh = plsc.VectorSubcoreMesh(core_axis_name="core",
                                  subcore_axis_name="subcore")

    @functools.partial(
        pl.kernel,
        out_type=jax.ShapeDtypeStruct((m, width), src.dtype),
        mesh=mesh)
    def kern(x_hbm, i_hbm, o_hbm):
        def body(i_vmem, o_vmem):
            pltpu.sync_copy(x_hbm.at[i_vmem.at[0]], o_vmem)

        pltpu.emit_pipeline(
            body,
            grid=(m // SC_W,),
            in_specs=[pl.BlockSpec((1, SC_W), index_map=lambda i: (0, i))],
            out_specs=[pl.BlockSpec((SC_W, width),
                                    index_map=lambda i: (i, 0))],
            core_axis_name=("core", "subcore"),
            dimension_semantics=(pltpu.PARALLEL,),
        )(i_hbm, o_hbm)

    return kern(src, idx2)


def _router_kernel(x_ref, gw_ref, i0_ref, i1_ref, w0_ref, w1_ref, psum_ref,
                   acc_ref):
    g = pl.program_id(0)
    logits = jax.lax.dot_general(
        x_ref[...].astype(jnp.bfloat16), gw_ref[...].astype(jnp.bfloat16),
        (((1,), (0,)), ((), ())),
        preferred_element_type=jnp.float32)
    m1 = jnp.max(logits, axis=-1, keepdims=True)
    i1 = jnp.argmax(logits, axis=-1).astype(jnp.int32)[:, None]
    lane = jax.lax.broadcasted_iota(jnp.int32, logits.shape, 1)
    masked = jnp.where(lane == i1, -jnp.inf, logits)
    m2 = jnp.max(masked, axis=-1, keepdims=True)
    i2 = jnp.argmax(masked, axis=-1).astype(jnp.int32)[:, None]
    e2 = jnp.exp(m2 - m1)
    i0_ref[...] = i1
    i1_ref[...] = i2
    w0_ref[...] = 1.0 / (1.0 + e2)
    w1_ref[...] = e2 / (1.0 + e2)
    p = jnp.exp(logits - m1)
    p = p / jnp.sum(p, axis=-1, keepdims=True)

    @pl.when(g == 0)
    def _():
        acc_ref[...] = jnp.zeros_like(acc_ref)

    acc_ref[...] += jnp.sum(p, axis=0, keepdims=True)

    @pl.when(g == pl.num_programs(0) - 1)
    def _():
        psum_ref[...] = acc_ref[...]


def _stage1_kernel(sp_ref, xs_ref, wg_ref, wu_ref, h_ref, *, nb):
    g = pl.program_id(1)
    nvalid = sp_ref[nb]

    @pl.when(g < nvalid)
    def _():
        xs = xs_ref[...]
        g1 = jax.lax.dot_general(xs, wg_ref[0], (((1,), (0,)), ((), ())),
                                 precision=jax.lax.Precision.DEFAULT,
                                 preferred_element_type=jnp.float32)
        g2 = jax.lax.dot_general(xs, wu_ref[0], (((1,), (0,)), ((), ())),
                                 precision=jax.lax.Precision.DEFAULT,
                                 preferred_element_type=jnp.float32)
        h_ref[...] = g1 * jax.nn.sigmoid(g1) * g2


def _stage2_kernel(sp_ref, h_ref, wcol_ref, wd_ref, y_ref, *, nb):
    g = pl.program_id(0)
    nvalid = sp_ref[nb]

    @pl.when(g < nvalid)
    def _():
        y = jax.lax.dot_general(h_ref[...], wd_ref[0], (((1,), (0,)), ((), ())),
                                precision=jax.lax.Precision.DEFAULT,
                                preferred_element_type=jnp.float32)
        y_ref[...] = (y * wcol_ref[...]).astype(jnp.bfloat16)


def _sum2_kernel(y3_ref, o_ref):
    o_ref[...] = jnp.sum(y3_ref[...].astype(jnp.float32), axis=1)


def kernel(x, gate_w, w_gate, w_up, w_down):
    b, s, d = x.shape
    e_num = gate_w.shape[1]
    f = w_gate.shape[2]
    t = b * s
    a = t * TOP_K
    a_pad = a + e_num * BM
    nb = a_pad // BM

    xf = x.reshape(t, d)

    bmr = min(512, t)
    i0, i1, w0, w1, psum = pl.pallas_call(
        _router_kernel,
        grid=(t // bmr,),
        in_specs=[
            pl.BlockSpec((bmr, d), lambda g: (g, 0)),
            pl.BlockSpec((d, e_num), lambda g: (0, 0)),
        ],
        out_specs=[
            pl.BlockSpec((bmr, 1), lambda g: (g, 0)),
            pl.BlockSpec((bmr, 1), lambda g: (g, 0)),
            pl.BlockSpec((bmr, 1), lambda g: (g, 0)),
            pl.BlockSpec((bmr, 1), lambda g: (g, 0)),
            pl.BlockSpec((1, e_num), lambda g: (0, 0)),
        ],
        out_shape=[
            jax.ShapeDtypeStruct((t, 1), jnp.int32),
            jax.ShapeDtypeStruct((t, 1), jnp.int32),
            jax.ShapeDtypeStruct((t, 1), jnp.float32),
            jax.ShapeDtypeStruct((t, 1), jnp.float32),
            jax.ShapeDtypeStruct((1, e_num), jnp.float32),
        ],
        scratch_shapes=[pltpu.VMEM((1, e_num), jnp.float32)],
        compiler_params=pltpu.CompilerParams(
            dimension_semantics=("arbitrary",)),
    )(xf, gate_w)

    idx = jnp.concatenate([i0, i1], axis=1)
    rw = jnp.concatenate([w0, w1], axis=1)
    e_a = idx.reshape(-1)
    onehot = (e_a[:, None] == jnp.arange(e_num, dtype=jnp.int32)[None, :])
    cum = jnp.cumsum(onehot.astype(jnp.int32), axis=0)
    rank = jnp.take_along_axis(cum, e_a[:, None], axis=1)[:, 0] - 1
    counts = cum[-1]
    padded = ((counts + BM - 1) // BM) * BM
    ends = jnp.cumsum(padded)
    offsets = ends - padded
    pos = (offsets[e_a] + rank).astype(jnp.int32)
    tok = (jnp.arange(a, dtype=jnp.int32) // TOP_K)
    tok_of_pos = jnp.zeros((a_pad,), jnp.int32).at[pos].set(tok)
    w_of_pos = jnp.zeros((a_pad,), jnp.float32).at[pos].set(rw.reshape(-1))
    nvalid = (ends[-1] // BM).astype(jnp.int32)
    blk_start = jnp.arange(nb, dtype=jnp.int32) * BM
    block_expert = jnp.searchsorted(ends, blk_start, side='right')
    block_expert = jnp.minimum(block_expert, e_num - 1).astype(jnp.int32)
    sp = jnp.concatenate([block_expert, nvalid[None]])

    lane_d = min(256, d)
    sub = d // lane_d
    sub_i = jnp.arange(sub, dtype=jnp.int32)
    idx_disp = (tok_of_pos[:, None] * sub + sub_i).reshape(-1)
    xs = _sc_gather(xf.reshape(t * sub, lane_d), idx_disp).reshape(a_pad, d)

    nj = 4
    fc = f // nj
    h = pl.pallas_call(
        functools.partial(_stage1_kernel, nb=nb),
        grid_spec=pltpu.PrefetchScalarGridSpec(
            num_scalar_prefetch=1,
            grid=(nj, nb),
            in_specs=[
                pl.BlockSpec((BM, d), lambda j, g, sp_: (g, 0)),
                pl.BlockSpec((1, d, fc), lambda j, g, sp_: (sp_[g], 0, j)),
                pl.BlockSpec((1, d, fc), lambda j, g, sp_: (sp_[g], 0, j)),
            ],
            out_specs=pl.BlockSpec((BM, fc), lambda j, g, sp_: (g, j)),
        ),
        out_shape=jax.ShapeDtypeStruct((a_pad, f), jnp.float32),
        compiler_params=pltpu.CompilerParams(
            dimension_semantics=("arbitrary", "arbitrary")),
    )(sp, xs, w_gate, w_up)

    y = pl.pallas_call(
        functools.partial(_stage2_kernel, nb=nb),
        grid_spec=pltpu.PrefetchScalarGridSpec(
            num_scalar_prefetch=1,
            grid=(nb,),
            in_specs=[
                pl.BlockSpec((BM, f), lambda g, sp_: (g, 0)),
                pl.BlockSpec((BM, 1), lambda g, sp_: (g, 0)),
                pl.BlockSpec((1, f, d), lambda g, sp_: (sp_[g], 0, 0)),
            ],
            out_specs=pl.BlockSpec((BM, d), lambda g, sp_: (g, 0)),
        ),
        out_shape=jax.ShapeDtypeStruct((a_pad, d), jnp.bfloat16),
        compiler_params=pltpu.CompilerParams(
            dimension_semantics=("arbitrary",)),
    )(sp, h, w_of_pos[:, None], w_down)

    y32 = jax.lax.bitcast_convert_type(
        y.reshape(a_pad, d // 2, 2), jnp.int32)
    lane_c = min(256, d // 2)
    sub_c = d // 2 // lane_c
    sub_ci = jnp.arange(sub_c, dtype=jnp.int32)
    idx_comb = (pos[:, None] * sub_c + sub_ci).reshape(-1)
    y01_32 = _sc_gather(y32.reshape(a_pad * sub_c, lane_c), idx_comb)
    y3 = jax.lax.bitcast_convert_type(
        y01_32.reshape(t, TOP_K, d // 2), jnp.bfloat16).reshape(t, TOP_K, d)
    bmc = min(512, t)
    out = pl.pallas_call(
        _sum2_kernel,
        grid=(t // bmc,),
        in_specs=[pl.BlockSpec((bmc, TOP_K, d), lambda g: (g, 0, 0))],
        out_specs=pl.BlockSpec((bmc, d), lambda g: (g, 0)),
        out_shape=jax.ShapeDtypeStruct((t, d), jnp.float32),
    )(y3)

    output = out.reshape(b, s, d)
    probs_mean = psum[0] / t
    frac = counts.astype(jnp.float32) / a
    aux = jnp.sum(probs_mean * frac) * e_num
    return output, aux


# speedup vs baseline: 1.3593x; 1.3557x over previous
import functools

import jax
import jax.numpy as jnp
from jax.experimental import pallas as pl
from jax.experimental.pallas import tpu as pltpu

TOP_K = 2
BM = 128
NJ = 4


def _router_kernel(x_ref, gw_ref, i0_ref, i1_ref, w0_ref, w1_ref, psum_ref,
                   acc_ref):
    g = pl.program_id(0)
    logits = jax.lax.dot_general(
        x_ref[...].astype(jnp.bfloat16), gw_ref[...].astype(jnp.bfloat16),
        (((1,), (0,)), ((), ())),
        preferred_element_type=jnp.float32)
    m1 = jnp.max(logits, axis=-1, keepdims=True)
    i1 = jnp.argmax(logits, axis=-1).astype(jnp.int32)[:, None]
    lane = jax.lax.broadcasted_iota(jnp.int32, logits.shape, 1)
    masked = jnp.where(lane == i1, -jnp.inf, logits)
    m2 = jnp.max(masked, axis=-1, keepdims=True)
    i2 = jnp.argmax(masked, axis=-1).astype(jnp.int32)[:, None]
    e2 = jnp.exp(m2 - m1)
    i0_ref[...] = i1
    i1_ref[...] = i2
    w0_ref[...] = 1.0 / (1.0 + e2)
    w1_ref[...] = e2 / (1.0 + e2)
    p = jnp.exp(logits - m1)
    p = p / jnp.sum(p, axis=-1, keepdims=True)

    @pl.when(g == 0)
    def _():
        acc_ref[...] = jnp.zeros_like(acc_ref)

    acc_ref[...] += jnp.sum(p, axis=0, keepdims=True)

    @pl.when(g == pl.num_programs(0) - 1)
    def _():
        psum_ref[...] = acc_ref[...]


def _dispatch_kernel(sp_ref, xb_ref, tok_ref, xs_ref, *, nb, t):
    g = pl.program_id(0)
    nvalid = sp_ref[nb]

    @pl.when(g < nvalid)
    def _():
        tok = tok_ref[...]
        col = jax.lax.broadcasted_iota(jnp.int32, (BM, t), 1)
        oh = jnp.where(col == tok, 1.0, 0.0).astype(jnp.bfloat16)
        xs = jax.lax.dot_general(
            oh, xb_ref[...], (((1,), (0,)), ((), ())),
            preferred_element_type=jnp.float32)
        xs_ref[...] = xs.astype(jnp.bfloat16)


def _stage1_kernel(sp_ref, xs_ref, wg_ref, wu_ref, h_ref, wg_bf, wu_bf,
                   flag_ref, *, nb):
    j = pl.program_id(0)
    g = pl.program_id(1)
    nvalid = sp_ref[nb]
    cid = sp_ref[g] * NJ + j

    @pl.when(((j == 0) & (g == 0)) | (cid != flag_ref[0]))
    def _():
        wg_bf[...] = wg_ref[0].astype(jnp.bfloat16)
        wu_bf[...] = wu_ref[0].astype(jnp.bfloat16)
        flag_ref[0] = cid

    @pl.when(g < nvalid)
    def _():
        xs = xs_ref[...]
        g1 = jax.lax.dot_general(xs, wg_bf[...], (((1,), (0,)), ((), ())),
                                 preferred_element_type=jnp.float32)
        g2 = jax.lax.dot_general(xs, wu_bf[...], (((1,), (0,)), ((), ())),
                                 preferred_element_type=jnp.float32)
        h_ref[...] = (g1 * jax.nn.sigmoid(g1) * g2).astype(jnp.bfloat16)


def _stage2_kernel(sp_ref, h_ref, wcol_ref, wd_ref, y_ref, wd_bf, flag_ref, *,
                   nb):
    g = pl.program_id(0)
    nvalid = sp_ref[nb]
    cid = sp_ref[g]

    @pl.when((g == 0) | (cid != flag_ref[0]))
    def _():
        wd_bf[...] = wd_ref[0].astype(jnp.bfloat16)
        flag_ref[0] = cid

    @pl.when(g < nvalid)
    def _():
        y = jax.lax.dot_general(h_ref[...], wd_bf[...],
                                (((1,), (0,)), ((), ())),
                                preferred_element_type=jnp.float32)
        y_ref[...] = (y * wcol_ref[...]).astype(jnp.bfloat16)


def _combine_kernel(p0_ref, p1_ref, y_ref, o_ref, *, a_pad):
    p0 = p0_ref[...]
    p1 = p1_ref[...]
    bmc = p0.shape[0]
    col = jax.lax.broadcasted_iota(jnp.int32, (bmc, a_pad), 1)
    oh = ((col == p0) | (col == p1)).astype(jnp.bfloat16)
    o_ref[...] = jax.lax.dot_general(
        oh, y_ref[...], (((1,), (0,)), ((), ())),
        preferred_element_type=jnp.float32)


def kernel(x, gate_w, w_gate, w_up, w_down):
    b, s, d = x.shape
    e_num = gate_w.shape[1]
    f = w_gate.shape[2]
    t = b * s
    a = t * TOP_K
    a_pad = a + e_num * BM
    nb = a_pad // BM
    fc = f // NJ

    xf = x.reshape(t, d)
    xb = xf.astype(jnp.bfloat16)

    bmr = min(512, t)
    i0, i1, w0, w1, psum = pl.pallas_call(
        _router_kernel,
        grid=(t // bmr,),
        in_specs=[
            pl.BlockSpec((bmr, d), lambda g: (g, 0)),
            pl.BlockSpec((d, e_num), lambda g: (0, 0)),
        ],
        out_specs=[
            pl.BlockSpec((bmr, 1), lambda g: (g, 0)),
            pl.BlockSpec((bmr, 1), lambda g: (g, 0)),
            pl.BlockSpec((bmr, 1), lambda g: (g, 0)),
            pl.BlockSpec((bmr, 1), lambda g: (g, 0)),
            pl.BlockSpec((1, e_num), lambda g: (0, 0)),
        ],
        out_shape=[
            jax.ShapeDtypeStruct((t, 1), jnp.int32),
            jax.ShapeDtypeStruct((t, 1), jnp.int32),
            jax.ShapeDtypeStruct((t, 1), jnp.float32),
            jax.ShapeDtypeStruct((t, 1), jnp.float32),
            jax.ShapeDtypeStruct((1, e_num), jnp.float32),
        ],
        scratch_shapes=[pltpu.VMEM((1, e_num), jnp.float32)],
        compiler_params=pltpu.CompilerParams(
            dimension_semantics=("arbitrary",)),
    )(xf, gate_w)

    idx = jnp.concatenate([i0, i1], axis=1)
    rw = jnp.concatenate([w0, w1], axis=1)
    e_a = idx.reshape(-1)
    onehot = (e_a[:, None] == jnp.arange(e_num, dtype=jnp.int32)[None, :])
    cum = jnp.cumsum(onehot.astype(jnp.int32), axis=0)
    rank = jnp.take_along_axis(cum, e_a[:, None], axis=1)[:, 0] - 1
    counts = cum[-1]
    padded = ((counts + BM - 1) // BM) * BM
    ends = jnp.cumsum(padded)
    offsets = ends - padded
    pos = (offsets[e_a] + rank).astype(jnp.int32)
    tok = (jnp.arange(a, dtype=jnp.int32) // TOP_K)
    tok_of_pos = jnp.zeros((a_pad,), jnp.int32).at[pos].set(tok)
    w_of_pos = jnp.zeros((a_pad,), jnp.float32).at[pos].set(rw.reshape(-1))
    nvalid = (ends[-1] // BM).astype(jnp.int32)
    blk_start = jnp.arange(nb, dtype=jnp.int32) * BM
    block_expert = jnp.searchsorted(ends, blk_start, side='right')
    block_expert = jnp.minimum(block_expert, e_num - 1).astype(jnp.int32)
    sp = jnp.concatenate([block_expert, nvalid[None]])

    xs = pl.pallas_call(
        functools.partial(_dispatch_kernel, nb=nb, t=t),
        grid_spec=pltpu.PrefetchScalarGridSpec(
            num_scalar_prefetch=1,
            grid=(nb,),
            in_specs=[
                pl.BlockSpec((t, d), lambda g, sp_: (0, 0)),
                pl.BlockSpec((BM, 1), lambda g, sp_: (g, 0)),
            ],
            out_specs=pl.BlockSpec((BM, d), lambda g, sp_: (g, 0)),
        ),
        out_shape=jax.ShapeDtypeStruct((a_pad, d), jnp.bfloat16),
        compiler_params=pltpu.CompilerParams(
            dimension_semantics=("arbitrary",)),
    )(sp, xb, tok_of_pos[:, None])

    h = pl.pallas_call(
        functools.partial(_stage1_kernel, nb=nb),
        grid_spec=pltpu.PrefetchScalarGridSpec(
            num_scalar_prefetch=1,
            grid=(NJ, nb),
            in_specs=[
                pl.BlockSpec((BM, d), lambda j, g, sp_: (g, 0)),
                pl.BlockSpec((1, d, fc), lambda j, g, sp_: (sp_[g], 0, j)),
                pl.BlockSpec((1, d, fc), lambda j, g, sp_: (sp_[g], 0, j)),
            ],
            out_specs=pl.BlockSpec((BM, fc), lambda j, g, sp_: (g, j)),
            scratch_shapes=[
                pltpu.VMEM((d, fc), jnp.bfloat16),
                pltpu.VMEM((d, fc), jnp.bfloat16),
                pltpu.SMEM((1,), jnp.int32),
            ],
        ),
        out_shape=jax.ShapeDtypeStruct((a_pad, f), jnp.bfloat16),
        compiler_params=pltpu.CompilerParams(
            dimension_semantics=("arbitrary", "arbitrary")),
    )(sp, xs, w_gate, w_up)

    y = pl.pallas_call(
        functools.partial(_stage2_kernel, nb=nb),
        grid_spec=pltpu.PrefetchScalarGridSpec(
            num_scalar_prefetch=1,
            grid=(nb,),
            in_specs=[
                pl.BlockSpec((BM, f), lambda g, sp_: (g, 0)),
                pl.BlockSpec((BM, 1), lambda g, sp_: (g, 0)),
                pl.BlockSpec((1, f, d), lambda g, sp_: (sp_[g], 0, 0)),
            ],
            out_specs=pl.BlockSpec((BM, d), lambda g, sp_: (g, 0)),
            scratch_shapes=[
                pltpu.VMEM((f, d), jnp.bfloat16),
                pltpu.SMEM((1,), jnp.int32),
            ],
        ),
        out_shape=jax.ShapeDtypeStruct((a_pad, d), jnp.bfloat16),
        compiler_params=pltpu.CompilerParams(
            dimension_semantics=("arbitrary",)),
    )(sp, h, w_of_pos[:, None], w_down)

    bmc = min(128, t)
    p0 = pos[0::TOP_K][:, None]
    p1 = pos[1::TOP_K][:, None]
    out = pl.pallas_call(
        functools.partial(_combine_kernel, a_pad=a_pad),
        grid=(t // bmc,),
        in_specs=[
            pl.BlockSpec((bmc, 1), lambda g: (g, 0)),
            pl.BlockSpec((bmc, 1), lambda g: (g, 0)),
            pl.BlockSpec((a_pad, d), lambda g: (0, 0)),
        ],
        out_specs=pl.BlockSpec((bmc, d), lambda g: (g, 0)),
        out_shape=jax.ShapeDtypeStruct((t, d), jnp.float32),
    )(p0, p1, y)

    output = out.reshape(b, s, d)
    probs_mean = psum[0] / t
    frac = counts.astype(jnp.float32) / a
    aux = jnp.sum(probs_mean * frac) * e_num
    return output, aux


# speedup vs baseline: 1.4285x; 1.0509x over previous
import functools

import jax
import jax.numpy as jnp
from jax.experimental import pallas as pl
from jax.experimental.pallas import tpu as pltpu

TOP_K = 2
BM = 128


def _router_kernel(x_ref, gw_ref, i0_ref, i1_ref, w0_ref, w1_ref, psum_ref,
                   acc_ref):
    g = pl.program_id(0)
    logits = jax.lax.dot_general(
        x_ref[...].astype(jnp.bfloat16), gw_ref[...].astype(jnp.bfloat16),
        (((1,), (0,)), ((), ())),
        preferred_element_type=jnp.float32)
    m1 = jnp.max(logits, axis=-1, keepdims=True)
    i1 = jnp.argmax(logits, axis=-1).astype(jnp.int32)[:, None]
    lane = jax.lax.broadcasted_iota(jnp.int32, logits.shape, 1)
    masked = jnp.where(lane == i1, -jnp.inf, logits)
    m2 = jnp.max(masked, axis=-1, keepdims=True)
    i2 = jnp.argmax(masked, axis=-1).astype(jnp.int32)[:, None]
    e2 = jnp.exp(m2 - m1)
    i0_ref[...] = i1
    i1_ref[...] = i2
    w0_ref[...] = 1.0 / (1.0 + e2)
    w1_ref[...] = e2 / (1.0 + e2)
    p = jnp.exp(logits - m1)
    p = p / jnp.sum(p, axis=-1, keepdims=True)

    @pl.when(g == 0)
    def _():
        acc_ref[...] = jnp.zeros_like(acc_ref)

    acc_ref[...] += jnp.sum(p, axis=0, keepdims=True)

    @pl.when(g == pl.num_programs(0) - 1)
    def _():
        psum_ref[...] = acc_ref[...]


def _ffn_kernel(sp_ref, xb_ref, tok_ref, wcol_ref, wg_hbm, wu_hbm, wd_hbm,
                y_ref, wg_v, wu_v, wd_v, sems, *, nb, t):
    g = pl.program_id(0)
    e = sp_ref[g]
    e_prev = sp_ref[jnp.maximum(g - 1, 0)]
    nvalid = sp_ref[nb]

    @pl.when((g == 0) | (e != e_prev))
    def _():
        c1 = pltpu.make_async_copy(wg_hbm.at[e], wg_v, sems.at[0])
        c2 = pltpu.make_async_copy(wu_hbm.at[e], wu_v, sems.at[1])
        c3 = pltpu.make_async_copy(wd_hbm.at[e], wd_v, sems.at[2])
        c1.start()
        c2.start()
        c3.start()
        c1.wait()
        c2.wait()
        c3.wait()

    @pl.when(g < nvalid)
    def _():
        tok = tok_ref[...]
        col = jax.lax.broadcasted_iota(jnp.int32, (BM, t), 1)
        oh = jnp.where(col == tok, 1.0, 0.0).astype(jnp.bfloat16)
        xs = jax.lax.dot_general(
            oh, xb_ref[...], (((1,), (0,)), ((), ())),
            preferred_element_type=jnp.float32).astype(jnp.bfloat16)
        g1 = jax.lax.dot_general(xs, wg_v[...], (((1,), (0,)), ((), ())),
                                 preferred_element_type=jnp.float32)
        g2 = jax.lax.dot_general(xs, wu_v[...], (((1,), (0,)), ((), ())),
                                 preferred_element_type=jnp.float32)
        h = (g1 * jax.nn.sigmoid(g1) * g2).astype(jnp.bfloat16)
        y = jax.lax.dot_general(h, wd_v[...], (((1,), (0,)), ((), ())),
                                preferred_element_type=jnp.float32)
        y_ref[...] = (y * wcol_ref[...]).astype(jnp.bfloat16)


def _combine_kernel(p0_ref, p1_ref, y_ref, o_ref, *, a_pad):
    p0 = p0_ref[...]
    p1 = p1_ref[...]
    bmc = p0.shape[0]
    col = jax.lax.broadcasted_iota(jnp.int32, (bmc, a_pad), 1)
    oh = ((col == p0) | (col == p1)).astype(jnp.bfloat16)
    o_ref[...] = jax.lax.dot_general(
        oh, y_ref[...], (((1,), (0,)), ((), ())),
        preferred_element_type=jnp.float32)


def kernel(x, gate_w, w_gate, w_up, w_down):
    b, s, d = x.shape
    e_num = gate_w.shape[1]
    f = w_gate.shape[2]
    t = b * s
    a = t * TOP_K
    a_pad = a + e_num * BM
    nb = a_pad // BM

    xf = x.reshape(t, d)
    xb = xf.astype(jnp.bfloat16)
    wg = w_gate.astype(jnp.bfloat16)
    wu = w_up.astype(jnp.bfloat16)
    wd = w_down.astype(jnp.bfloat16)

    bmr = min(512, t)
    i0, i1, w0, w1, psum = pl.pallas_call(
        _router_kernel,
        grid=(t // bmr,),
        in_specs=[
            pl.BlockSpec((bmr, d), lambda g: (g, 0)),
            pl.BlockSpec((d, e_num), lambda g: (0, 0)),
        ],
        out_specs=[
            pl.BlockSpec((bmr, 1), lambda g: (g, 0)),
            pl.BlockSpec((bmr, 1), lambda g: (g, 0)),
            pl.BlockSpec((bmr, 1), lambda g: (g, 0)),
            pl.BlockSpec((bmr, 1), lambda g: (g, 0)),
            pl.BlockSpec((1, e_num), lambda g: (0, 0)),
        ],
        out_shape=[
            jax.ShapeDtypeStruct((t, 1), jnp.int32),
            jax.ShapeDtypeStruct((t, 1), jnp.int32),
            jax.ShapeDtypeStruct((t, 1), jnp.float32),
            jax.ShapeDtypeStruct((t, 1), jnp.float32),
            jax.ShapeDtypeStruct((1, e_num), jnp.float32),
        ],
        scratch_shapes=[pltpu.VMEM((1, e_num), jnp.float32)],
        compiler_params=pltpu.CompilerParams(
            dimension_semantics=("arbitrary",)),
    )(xf, gate_w)

    idx = jnp.concatenate([i0, i1], axis=1)
    rw = jnp.concatenate([w0, w1], axis=1)
    e_a = idx.reshape(-1)
    onehot = (e_a[:, None] == jnp.arange(e_num, dtype=jnp.int32)[None, :])
    cum = jnp.cumsum(onehot.astype(jnp.int32), axis=0)
    rank = jnp.take_along_axis(cum, e_a[:, None], axis=1)[:, 0] - 1
    counts = cum[-1]
    padded = ((counts + BM - 1) // BM) * BM
    ends = jnp.cumsum(padded)
    offsets = ends - padded
    pos = (offsets[e_a] + rank).astype(jnp.int32)
    tok = (jnp.arange(a, dtype=jnp.int32) // TOP_K)
    tok_of_pos = jnp.zeros((a_pad,), jnp.int32).at[pos].set(tok)
    w_of_pos = jnp.zeros((a_pad,), jnp.float32).at[pos].set(rw.reshape(-1))
    nvalid = (ends[-1] // BM).astype(jnp.int32)
    blk_start = jnp.arange(nb, dtype=jnp.int32) * BM
    block_expert = jnp.searchsorted(ends, blk_start, side='right')
    block_expert = jnp.minimum(block_expert, e_num - 1).astype(jnp.int32)
    sp = jnp.concatenate([block_expert, nvalid[None]])

    grid_spec = pltpu.PrefetchScalarGridSpec(
        num_scalar_prefetch=1,
        grid=(nb,),
        in_specs=[
            pl.BlockSpec((t, d), lambda g, sp_: (0, 0)),
            pl.BlockSpec((BM, 1), lambda g, sp_: (g, 0)),
            pl.BlockSpec((BM, 1), lambda g, sp_: (g, 0)),
            pl.BlockSpec(memory_space=pl.ANY),
            pl.BlockSpec(memory_space=pl.ANY),
            pl.BlockSpec(memory_space=pl.ANY),
        ],
        out_specs=pl.BlockSpec((BM, d), lambda g, sp_: (g, 0)),
        scratch_shapes=[
            pltpu.VMEM((d, f), jnp.bfloat16),
            pltpu.VMEM((d, f), jnp.bfloat16),
            pltpu.VMEM((f, d), jnp.bfloat16),
            pltpu.SemaphoreType.DMA((3,)),
        ],
    )
    y = pl.pallas_call(
        functools.partial(_ffn_kernel, nb=nb, t=t),
        grid_spec=grid_spec,
        out_shape=jax.ShapeDtypeStruct((a_pad, d), jnp.bfloat16),
        compiler_params=pltpu.CompilerParams(
            dimension_semantics=("arbitrary",)),
    )(sp, xb, tok_of_pos[:, None], w_of_pos[:, None], wg, wu, wd)

    bmc = min(128, t)
    p0 = pos[0::TOP_K][:, None]
    p1 = pos[1::TOP_K][:, None]
    out = pl.pallas_call(
        functools.partial(_combine_kernel, a_pad=a_pad),
        grid=(t // bmc,),
        in_specs=[
            pl.BlockSpec((bmc, 1), lambda g: (g, 0)),
            pl.BlockSpec((bmc, 1), lambda g: (g, 0)),
            pl.BlockSpec((a_pad, d), lambda g: (0, 0)),
        ],
        out_specs=pl.BlockSpec((bmc, d), lambda g: (g, 0)),
        out_shape=jax.ShapeDtypeStruct((t, d), jnp.float32),
    )(p0, p1, y)

    output = out.reshape(b, s, d)
    probs_mean = psum[0] / t
    frac = counts.astype(jnp.float32) / a
    aux = jnp.sum(probs_mean * frac) * e_num
    return output, aux


# speedup vs baseline: 1.4524x; 1.0168x over previous
import functools

import jax
import jax.numpy as jnp
from jax.experimental import pallas as pl
from jax.experimental.pallas import tpu as pltpu

TOP_K = 2
BM = 128


def _router_kernel(x_ref, gw_ref, i0_ref, i1_ref, w0_ref, w1_ref, psum_ref,
                   acc_ref):
    g = pl.program_id(0)
    logits = jax.lax.dot_general(
        x_ref[...].astype(jnp.bfloat16), gw_ref[...].astype(jnp.bfloat16),
        (((1,), (0,)), ((), ())),
        preferred_element_type=jnp.float32)
    m1 = jnp.max(logits, axis=-1, keepdims=True)
    i1 = jnp.argmax(logits, axis=-1).astype(jnp.int32)[:, None]
    lane = jax.lax.broadcasted_iota(jnp.int32, logits.shape, 1)
    masked = jnp.where(lane == i1, -jnp.inf, logits)
    m2 = jnp.max(masked, axis=-1, keepdims=True)
    i2 = jnp.argmax(masked, axis=-1).astype(jnp.int32)[:, None]
    e2 = jnp.exp(m2 - m1)
    i0_ref[...] = i1
    i1_ref[...] = i2
    w0_ref[...] = 1.0 / (1.0 + e2)
    w1_ref[...] = e2 / (1.0 + e2)
    p = jnp.exp(logits - m1)
    p = p / jnp.sum(p, axis=-1, keepdims=True)

    @pl.when(g == 0)
    def _():
        acc_ref[...] = jnp.zeros_like(acc_ref)

    acc_ref[...] += jnp.sum(p, axis=0, keepdims=True)

    @pl.when(g == pl.num_programs(0) - 1)
    def _():
        psum_ref[...] = acc_ref[...]


def _ffn_kernel(sp_ref, xb_ref, tok_ref, wcol_ref, wg_hbm, wu_hbm, wd_hbm,
                y_ref, wg_v, wu_v, wd_v, sems, *, nb, t):
    g = pl.program_id(0)
    e = sp_ref[g]
    e_prev = sp_ref[jnp.maximum(g - 1, 0)]
    nvalid = sp_ref[nb]

    @pl.when((g == 0) | (e != e_prev))
    def _():
        c1 = pltpu.make_async_copy(wg_hbm.at[e], wg_v, sems.at[0])
        c2 = pltpu.make_async_copy(wu_hbm.at[e], wu_v, sems.at[1])
        c3 = pltpu.make_async_copy(wd_hbm.at[e], wd_v, sems.at[2])
        c1.start()
        c2.start()
        c3.start()
        c1.wait()
        c2.wait()
        c3.wait()

    @pl.when(g < nvalid)
    def _():
        tok = tok_ref[...]
        col = jax.lax.broadcasted_iota(jnp.int32, (BM, t), 1)
        oh = jnp.where(col == tok, 1.0, 0.0).astype(jnp.bfloat16)
        xs = jax.lax.dot_general(
            oh, xb_ref[...], (((1,), (0,)), ((), ())),
            preferred_element_type=jnp.float32).astype(jnp.bfloat16)
        g1 = jax.lax.dot_general(xs, wg_v[...], (((1,), (0,)), ((), ())),
                                 preferred_element_type=jnp.float32)
        g2 = jax.lax.dot_general(xs, wu_v[...], (((1,), (0,)), ((), ())),
                                 preferred_element_type=jnp.float32)
        h = (g1 * jax.nn.sigmoid(g1) * g2).astype(jnp.bfloat16)
        y = jax.lax.dot_general(h, wd_v[...], (((1,), (0,)), ((), ())),
                                preferred_element_type=jnp.float32)
        y_ref[...] = (y * wcol_ref[...]).astype(jnp.bfloat16)


def _combine_kernel(p0_ref, p1_ref, y_ref, o_ref, *, a_pad):
    p0 = p0_ref[...]
    p1 = p1_ref[...]
    bmc = p0.shape[0]
    col = jax.lax.broadcasted_iota(jnp.int32, (bmc, a_pad), 1)
    oh = ((col == p0) | (col == p1)).astype(jnp.bfloat16)
    o_ref[...] = jax.lax.dot_general(
        oh, y_ref[...], (((1,), (0,)), ((), ())),
        preferred_element_type=jnp.float32)


def kernel(x, gate_w, w_gate, w_up, w_down):
    b, s, d = x.shape
    e_num = gate_w.shape[1]
    f = w_gate.shape[2]
    t = b * s
    a = t * TOP_K
    a_pad = a + e_num * BM
    nb = a_pad // BM

    xf = x.reshape(t, d)
    xb = xf.astype(jnp.bfloat16)
    wg = w_gate.astype(jnp.bfloat16)
    wu = w_up.astype(jnp.bfloat16)
    wd = w_down.astype(jnp.bfloat16)

    bmr = min(512, t)
    i0, i1, w0, w1, psum = pl.pallas_call(
        _router_kernel,
        grid=(t // bmr,),
        in_specs=[
            pl.BlockSpec((bmr, d), lambda g: (g, 0)),
            pl.BlockSpec((d, e_num), lambda g: (0, 0)),
        ],
        out_specs=[
            pl.BlockSpec((bmr, 1), lambda g: (g, 0)),
            pl.BlockSpec((bmr, 1), lambda g: (g, 0)),
            pl.BlockSpec((bmr, 1), lambda g: (g, 0)),
            pl.BlockSpec((bmr, 1), lambda g: (g, 0)),
            pl.BlockSpec((1, e_num), lambda g: (0, 0)),
        ],
        out_shape=[
            jax.ShapeDtypeStruct((t, 1), jnp.int32),
            jax.ShapeDtypeStruct((t, 1), jnp.int32),
            jax.ShapeDtypeStruct((t, 1), jnp.float32),
            jax.ShapeDtypeStruct((t, 1), jnp.float32),
            jax.ShapeDtypeStruct((1, e_num), jnp.float32),
        ],
        scratch_shapes=[pltpu.VMEM((1, e_num), jnp.float32)],
        compiler_params=pltpu.CompilerParams(
            dimension_semantics=("arbitrary",)),
    )(xf, gate_w)

    idx = jnp.concatenate([i0, i1], axis=1)
    rw = jnp.concatenate([w0, w1], axis=1)
    e_a = idx.reshape(-1)
    onehot = (e_a[:, None] == jnp.arange(e_num, dtype=jnp.int32)[None, :])
    cum = jnp.cumsum(onehot.astype(jnp.int32), axis=0)
    rank = jnp.take_along_axis(cum, e_a[:, None], axis=1)[:, 0] - 1
    counts = cum[-1]
    padded = ((counts + BM - 1) // BM) * BM
    ends = jnp.cumsum(padded)
    offsets = ends - padded
    pos = (offsets[e_a] + rank).astype(jnp.int32)
    tok = (jnp.arange(a, dtype=jnp.int32) // TOP_K)
    tok_of_pos = jnp.zeros((a_pad,), jnp.int32).at[pos].set(tok)
    w_of_pos = jnp.zeros((a_pad,), jnp.float32).at[pos].set(rw.reshape(-1))
    nvalid = (ends[-1] // BM).astype(jnp.int32)
    blk_start = jnp.arange(nb, dtype=jnp.int32) * BM
    block_expert = jnp.searchsorted(ends, blk_start, side='right')
    block_expert = jnp.minimum(block_expert, e_num - 1).astype(jnp.int32)
    sp = jnp.concatenate([block_expert, nvalid[None]])

    grid_spec = pltpu.PrefetchScalarGridSpec(
        num_scalar_prefetch=1,
        grid=(nb,),
        in_specs=[
            pl.BlockSpec((t, d), lambda g, sp_: (0, 0)),
            pl.BlockSpec((BM, 1), lambda g, sp_: (g, 0)),
            pl.BlockSpec((BM, 1), lambda g, sp_: (g, 0)),
            pl.BlockSpec(memory_space=pl.ANY),
            pl.BlockSpec(memory_space=pl.ANY),
            pl.BlockSpec(memory_space=pl.ANY),
        ],
        out_specs=pl.BlockSpec((BM, d), lambda g, sp_: (g, 0)),
        scratch_shapes=[
            pltpu.VMEM((d, f), jnp.bfloat16),
            pltpu.VMEM((d, f), jnp.bfloat16),
            pltpu.VMEM((f, d), jnp.bfloat16),
            pltpu.SemaphoreType.DMA((3,)),
        ],
    )
    y = pl.pallas_call(
        functools.partial(_ffn_kernel, nb=nb, t=t),
        grid_spec=grid_spec,
        out_shape=jax.ShapeDtypeStruct((a_pad, d), jnp.bfloat16),
        compiler_params=pltpu.CompilerParams(
            dimension_semantics=("arbitrary",)),
    )(sp, xb, tok_of_pos[:, None], w_of_pos[:, None], wg, wu, wd)

    bmc = min(512, t)
    p0 = pos[0::TOP_K][:, None]
    p1 = pos[1::TOP_K][:, None]
    out = pl.pallas_call(
        functools.partial(_combine_kernel, a_pad=a_pad),
        grid=(t // bmc,),
        in_specs=[
            pl.BlockSpec((bmc, 1), lambda g: (g, 0)),
            pl.BlockSpec((bmc, 1), lambda g: (g, 0)),
            pl.BlockSpec((a_pad, d), lambda g: (0, 0)),
        ],
        out_specs=pl.BlockSpec((bmc, d), lambda g: (g, 0)),
        out_shape=jax.ShapeDtypeStruct((t, d), jnp.float32),
    )(p0, p1, y)

    output = out.reshape(b, s, d)
    probs_mean = psum[0] / t
    frac = counts.astype(jnp.float32) / a
    aux = jnp.sum(probs_mean * frac) * e_num
    return output, aux
